# baseline (speedup 1.0000x reference)
import functools

import jax
import jax.numpy as jnp
from jax import lax
from jax.experimental import pallas as pl
from jax.experimental.pallas import tpu as pltpu
from jax.experimental.pallas import tpu_sc as plsc

N = 10000
E = 320000
D = 128
NP = 10240
NC = 2
NS = 16
NW = NC * NS
ROWS_PER_W = NP // NW
CH = 128
SCCH = 3200
NSC = E // SCCH
FLUSH = 4096
IB = 2048
OB = FLUSH + SCCH + IB + 128
EB = E + 9728
GRID = 20
BR = NP // GRID

_MESH = plsc.VectorSubcoreMesh(core_axis_name="c", subcore_axis_name="s")
_CP = pltpu.CompilerParams(needs_layout_passes=False)


@functools.partial(
    pl.kernel,
    mesh=_MESH,
    compiler_params=_CP,
    out_type=[
        jax.ShapeDtypeStruct((NW * EB,), jnp.int32),
        jax.ShapeDtypeStruct((NW * EB,), jnp.int32),
        jax.ShapeDtypeStruct((NW, 16), jnp.int32),
        jax.ShapeDtypeStruct((NP,), jnp.float32),
    ],
    scratch_types=[
        pltpu.VMEM((SCCH,), jnp.int32),
        pltpu.VMEM((SCCH,), jnp.int32),
        pltpu.VMEM((SCCH,), jnp.int32),
        pltpu.VMEM((SCCH,), jnp.int32),
        pltpu.VMEM((OB,), jnp.int32),
        pltpu.VMEM((OB,), jnp.int32),
        pltpu.VMEM((ROWS_PER_W,), jnp.float32),
        pltpu.VMEM((16,), jnp.int32),
        pltpu.SemaphoreType.DMA,
        pltpu.SemaphoreType.DMA,
    ],
)
def _bucket_kernel(src_hbm, dst_hbm, bsrc_hbm, bdst_hbm, cnt_hbm, deg_hbm,
                   sva, dva, svb, dvb, osrc, odst, hist, cbuf, sema, semb):
    c = lax.axis_index("c")
    s = lax.axis_index("s")
    w = c * NS + s
    lo = w * ROWS_PER_W
    ones16 = jnp.ones((16,), jnp.float32)

    def zhist(i, carry):
        hist[pl.ds(i * 16, 16)] = jnp.zeros((16,), jnp.float32)
        return carry

    lax.fori_loop(0, ROWS_PER_W // 16, zhist, 0)

    iota16 = lax.iota(jnp.int32, 16)

    def stage(i, sv, dv, sem):
        o = pl.multiple_of(i * SCCH, 8)
        pltpu.async_copy(src_hbm.at[pl.ds(o, SCCH)], sv, sem)
        pltpu.async_copy(dst_hbm.at[pl.ds(o, SCCH)], dv, sem)

    def drain(sv, dv, sem):
        pltpu.make_async_copy(src_hbm.at[pl.ds(0, SCCH)], sv, sem).wait()
        pltpu.make_async_copy(dst_hbm.at[pl.ds(0, SCCH)], dv, sem).wait()

    def flush(ot):
        off_v, tot = ot
        pltpu.sync_copy(osrc.at[pl.ds(0, FLUSH)],
                        bsrc_hbm.at[pl.ds(pl.multiple_of(w * EB + tot, 8), FLUSH)])
        pltpu.sync_copy(odst.at[pl.ds(0, FLUSH)],
                        bdst_hbm.at[pl.ds(pl.multiple_of(w * EB + tot, 8), FLUSH)])
        for j in range(SCCH // 16):
            osrc[pl.ds(j * 16, 16)] = osrc[pl.ds(FLUSH + j * 16, 16)]
            odst[pl.ds(j * 16, 16)] = odst[pl.ds(FLUSH + j * 16, 16)]
        return off_v - FLUSH, tot + FLUSH

    def compute(sv, dv, ot):
        def group4(g, off_v):
            parts = []
            for u in range(4):
                q = g * 4 + u
                d16 = dv[pl.ds(q * 16, 16)]
                s16 = sv[pl.ds(q * 16, 16)]
                dl = d16 - lo
                m = (dl >= 0) & (dl < ROWS_PER_W)
                plsc.addupdate_scatter(hist, [dl], ones16, mask=m)
                pc = plsc.all_reduce_population_count(m)
                cs = plsc.cumsum(m.astype(jnp.int32))
                parts.append((s16, dl, m, pc, cs))
            o = off_v
            for s16, dl, m, pc, cs in parts:
                pos = o + cs - 1
                plsc.store_scatter(osrc, [pos], s16, mask=m)
                plsc.store_scatter(odst, [pos], dl, mask=m)
                o = o + pc
            return o

        off_v, tot = ot
        off_v = lax.fori_loop(0, SCCH // 64, group4, off_v)
        return lax.cond(jnp.max(off_v) >= FLUSH, flush,
                        lambda x: x, (off_v, tot))

    stage(0, sva, dva, sema)

    def pair(ip, ot):
        i0 = ip * 2
        stage(i0 + 1, svb, dvb, semb)
        drain(sva, dva, sema)
        ot = compute(sva, dva, ot)

        @pl.when(ip < NSC // 2 - 1)
        def _():
            stage(i0 + 2, sva, dva, sema)

        drain(svb, dvb, semb)
        return compute(svb, dvb, ot)

    off_v, tot = lax.fori_loop(
        0, NSC // 2, pair, (jnp.zeros((16,), jnp.int32), jnp.int32(0)))

    for j in range(IB // 16):
        pos = off_v + j * 16 + iota16
        plsc.store_scatter(osrc, [pos], iota16 + j * 16)
        plsc.store_scatter(odst, [pos],
                           jnp.zeros((16,), jnp.int32) + ROWS_PER_W)
    pltpu.sync_copy(osrc, bsrc_hbm.at[pl.ds(pl.multiple_of(w * EB + tot, 8), OB)])
    pltpu.sync_copy(odst, bdst_hbm.at[pl.ds(pl.multiple_of(w * EB + tot, 8), OB)])
    cbuf[...] = off_v + tot
    pltpu.sync_copy(cbuf, cnt_hbm.at[w])
    pltpu.sync_copy(hist, deg_hbm.at[pl.ds(pl.multiple_of(lo, 8), ROWS_PER_W)])


@functools.partial(
    pl.kernel,
    mesh=_MESH,
    compiler_params=_CP,
    out_type=jax.ShapeDtypeStruct((NP, D), jnp.float32),
    scratch_types=[
        pltpu.VMEM((ROWS_PER_W + 8, D), jnp.float32),
        pltpu.VMEM((IB,), jnp.int32),
        pltpu.VMEM((IB,), jnp.int32),
        pltpu.VMEM((CH, D), jnp.float32),
        pltpu.VMEM((CH, D), jnp.float32),
        pltpu.VMEM((16,), jnp.int32),
        pltpu.SemaphoreType.DMA,
        pltpu.SemaphoreType.DMA,
        pltpu.SemaphoreType.DMA,
    ],
)
def _agg_kernel(g_hbm, bsrc_hbm, bdst_hbm, cnt_hbm, out_hbm,
                acc, sblk, dblk, rowsa, rowsb, cbuf, semi, sema, semb):
    c = lax.axis_index("c")
    s = lax.axis_index("s")
    w = c * NS + s
    base = w * EB

    def zacc(i, carry):
        acc[i >> 3, pl.ds((i & 7) * 16, 16)] = jnp.zeros((16,), jnp.float32)
        return carry

    lax.fori_loop(0, ROWS_PER_W * (D // 16), zacc, 0)

    pltpu.sync_copy(cnt_hbm.at[w], cbuf)
    cnt = jnp.max(cbuf[...])
    nblk = (cnt + IB - 1) >> 11
    iota16 = lax.iota(jnp.int32, 16)

    def gstart(ch, rows, sem):
        pltpu.async_copy(
            g_hbm.at[sblk.at[pl.ds(pl.multiple_of(ch * CH, 8), CH)]],
            rows, sem)

    def gwait(rows, sem):
        pltpu.make_async_copy(g_hbm.at[pl.ds(0, CH)], rows, sem).wait()

    def compute(ch, rows):
        def qq(j, carry2):
            o = ch * CH + j * 32
            dla = dblk[pl.ds(o, 16)]
            dlb = dblk[pl.ds(o + 16, 16)]
            ra = j * 32 + iota16
            rb = ra + 16
            for p in range(D):
                cp = jnp.full((16,), p & ~15, jnp.int32) | (
                    (iota16 + (p & 15)) & 15)
                va = plsc.load_gather(rows, [ra, cp])
                vb = plsc.load_gather(rows, [rb, cp])
                plsc.addupdate_scatter(acc, [dla, cp], va)
                plsc.addupdate_scatter(acc, [dlb, cp], vb)
            return carry2

        lax.fori_loop(0, CH // 32, qq, 0)

    def blk(b, carry):
        o = pl.multiple_of(base + b * IB, 8)
        pltpu.async_copy(bsrc_hbm.at[pl.ds(o, IB)], sblk, semi)
        pltpu.async_copy(bdst_hbm.at[pl.ds(o, IB)], dblk, semi)
        pltpu.make_async_copy(bsrc_hbm.at[pl.ds(0, IB)], sblk, semi).wait()
        pltpu.make_async_copy(bdst_hbm.at[pl.ds(0, IB)], dblk, semi).wait()
        gstart(0, rowsa, sema)

        def pairb(jp, carry2):
            c0 = jp * 2
            gstart(c0 + 1, rowsb, semb)
            gwait(rowsa, sema)
            compute(c0, rowsa)

            @pl.when(jp < (IB // CH) // 2 - 1)
            def _():
                gstart(c0 + 2, rowsa, sema)

            gwait(rowsb, semb)
            compute(c0 + 1, rowsb)
            return carry2

        lax.fori_loop(0, (IB // CH) // 2, pairb, 0)
        return carry

    lax.fori_loop(0, nblk, blk, 0)
    pltpu.sync_copy(acc.at[pl.ds(0, ROWS_PER_W)],
                    out_hbm.at[pl.ds(pl.multiple_of(w * ROWS_PER_W, 8), ROWS_PER_W)])


def _tc0_body(x_ref, w_ref, deg_ref, g_ref, dinv_ref):
    i = pl.program_id(0)
    d = deg_ref[0, pl.ds(i * BR, BR)] + 1.0
    dinv = lax.rsqrt(jnp.maximum(d, 1e-12))
    m = jnp.dot(x_ref[...], w_ref[...], preferred_element_type=jnp.float32)
    g_ref[...] = m * dinv[:, None]
    dinv_ref[0, pl.ds(i * BR, BR)] = dinv


def _tc0(xp, W1, deg):
    return pl.pallas_call(
        _tc0_body,
        grid=(GRID,),
        in_specs=[
            pl.BlockSpec((BR, D), lambda i: (i, 0)),
            pl.BlockSpec((D, D), lambda i: (0, 0)),
            pl.BlockSpec((1, NP), lambda i: (0, 0)),
        ],
        out_specs=[
            pl.BlockSpec((BR, D), lambda i: (i, 0)),
            pl.BlockSpec((1, NP), lambda i: (0, 0)),
        ],
        out_shape=[
            jax.ShapeDtypeStruct((NP, D), jnp.float32),
            jax.ShapeDtypeStruct((1, NP), jnp.float32),
        ],
    )(xp, W1, deg)


def _tcl_body(s_ref, g_ref, dinv_ref, b_ref, w_ref, o_ref):
    i = pl.program_id(0)
    dinv = dinv_ref[0, pl.ds(i * BR, BR)]
    t = (s_ref[...] + g_ref[...]) * dinv[:, None] + b_ref[0, :][None, :]
    h = jnp.maximum(t, 0.0)
    o_ref[...] = jnp.dot(h, w_ref[...],
                         preferred_element_type=jnp.float32) * dinv[:, None]


def _tcl(sp, g, dinv, b, W):
    return pl.pallas_call(
        _tcl_body,
        grid=(GRID,),
        in_specs=[
            pl.BlockSpec((BR, D), lambda i: (i, 0)),
            pl.BlockSpec((BR, D), lambda i: (i, 0)),
            pl.BlockSpec((1, NP), lambda i: (0, 0)),
            pl.BlockSpec((1, D), lambda i: (0, 0)),
            pl.BlockSpec((D, D), lambda i: (0, 0)),
        ],
        out_specs=pl.BlockSpec((BR, D), lambda i: (i, 0)),
        out_shape=jax.ShapeDtypeStruct((NP, D), jnp.float32),
    )(sp, g, dinv, b, W)


def _tcf_body(s_ref, g_ref, dinv_ref, b_ref, o_ref):
    i = pl.program_id(0)
    dinv = dinv_ref[0, pl.ds(i * BR, BR)]
    o_ref[...] = (s_ref[...] + g_ref[...]) * dinv[:, None] \
        + b_ref[0, :][None, :]


def _tcf(sp, g, dinv, b):
    return pl.pallas_call(
        _tcf_body,
        grid=(GRID,),
        in_specs=[
            pl.BlockSpec((BR, D), lambda i: (i, 0)),
            pl.BlockSpec((BR, D), lambda i: (i, 0)),
            pl.BlockSpec((1, NP), lambda i: (0, 0)),
            pl.BlockSpec((1, D), lambda i: (0, 0)),
        ],
        out_specs=pl.BlockSpec((BR, D), lambda i: (i, 0)),
        out_shape=jax.ShapeDtypeStruct((NP, D), jnp.float32),
    )(sp, g, dinv, b)


def kernel(x, edge_index, batch, W1, b1, W2, b2, W3, b3):
    src = edge_index[0]
    dst = edge_index[1]
    xp = jnp.zeros((NP, D), jnp.float32).at[:N].set(x)
    bsrc, bdst, cnt, deg = _bucket_kernel(src, dst)
    g1, dinv = _tc0(xp, W1, deg.reshape(1, NP))
    s1 = _agg_kernel(g1, bsrc, bdst, cnt)
    g2 = _tcl(s1, g1, dinv, b1.reshape(1, D), W2)
    s2 = _agg_kernel(g2, bsrc, bdst, cnt)
    g3 = _tcl(s2, g2, dinv, b2.reshape(1, D), W3)
    s3 = _agg_kernel(g3, bsrc, bdst, cnt)
    out = _tcf(s3, g3, dinv, b3.reshape(1, D))
    return out[:N]


# speedup vs baseline: 1.0262x; 1.0262x over previous
import functools

import jax
import jax.numpy as jnp
from jax import lax
from jax.experimental import pallas as pl
from jax.experimental.pallas import tpu as pltpu
from jax.experimental.pallas import tpu_sc as plsc

N = 10000
E = 320000
D = 128
NP = 10240
NC = 2
NS = 16
NW = NC * NS
ROWS_PER_W = NP // NW
CH = 128
SCCH = 3200
NSC = E // SCCH
FLUSH = 4096
IB = 1024
OB = FLUSH + SCCH + IB + 128
EB = E + 8448
GRID = 20
BR = NP // GRID

_MESH = plsc.VectorSubcoreMesh(core_axis_name="c", subcore_axis_name="s")
_CP = pltpu.CompilerParams(needs_layout_passes=False)


@functools.partial(
    pl.kernel,
    mesh=_MESH,
    compiler_params=_CP,
    out_type=[
        jax.ShapeDtypeStruct((NW * EB,), jnp.int32),
        jax.ShapeDtypeStruct((NW * EB,), jnp.int32),
        jax.ShapeDtypeStruct((NW, 16), jnp.int32),
        jax.ShapeDtypeStruct((NP,), jnp.float32),
    ],
    scratch_types=[
        pltpu.VMEM((SCCH,), jnp.int32),
        pltpu.VMEM((SCCH,), jnp.int32),
        pltpu.VMEM((SCCH,), jnp.int32),
        pltpu.VMEM((SCCH,), jnp.int32),
        pltpu.VMEM((OB,), jnp.int32),
        pltpu.VMEM((OB,), jnp.int32),
        pltpu.VMEM((ROWS_PER_W,), jnp.float32),
        pltpu.VMEM((16,), jnp.int32),
        pltpu.SemaphoreType.DMA,
        pltpu.SemaphoreType.DMA,
    ],
)
def _bucket_kernel(src_hbm, dst_hbm, bsrc_hbm, bdst_hbm, cnt_hbm, deg_hbm,
                   sva, dva, svb, dvb, osrc, odst, hist, cbuf, sema, semb):
    c = lax.axis_index("c")
    s = lax.axis_index("s")
    w = c * NS + s
    lo = w * ROWS_PER_W
    ones16 = jnp.ones((16,), jnp.float32)

    def zhist(i, carry):
        hist[pl.ds(i * 16, 16)] = jnp.zeros((16,), jnp.float32)
        return carry

    lax.fori_loop(0, ROWS_PER_W // 16, zhist, 0)

    iota16 = lax.iota(jnp.int32, 16)

    def stage(i, sv, dv, sem):
        o = pl.multiple_of(i * SCCH, 8)
        pltpu.async_copy(src_hbm.at[pl.ds(o, SCCH)], sv, sem)
        pltpu.async_copy(dst_hbm.at[pl.ds(o, SCCH)], dv, sem)

    def drain(sv, dv, sem):
        pltpu.make_async_copy(src_hbm.at[pl.ds(0, SCCH)], sv, sem).wait()
        pltpu.make_async_copy(dst_hbm.at[pl.ds(0, SCCH)], dv, sem).wait()

    def flush(ot):
        off_v, tot = ot
        pltpu.sync_copy(osrc.at[pl.ds(0, FLUSH)],
                        bsrc_hbm.at[pl.ds(pl.multiple_of(w * EB + tot, 8), FLUSH)])
        pltpu.sync_copy(odst.at[pl.ds(0, FLUSH)],
                        bdst_hbm.at[pl.ds(pl.multiple_of(w * EB + tot, 8), FLUSH)])
        for j in range(SCCH // 16):
            osrc[pl.ds(j * 16, 16)] = osrc[pl.ds(FLUSH + j * 16, 16)]
            odst[pl.ds(j * 16, 16)] = odst[pl.ds(FLUSH + j * 16, 16)]
        return off_v - FLUSH, tot + FLUSH

    def compute(sv, dv, ot):
        def group4(g, off_v):
            parts = []
            for u in range(4):
                q = g * 4 + u
                d16 = dv[pl.ds(q * 16, 16)]
                s16 = sv[pl.ds(q * 16, 16)]
                dl = d16 - lo
                m = (dl >= 0) & (dl < ROWS_PER_W)
                plsc.addupdate_scatter(hist, [dl], ones16, mask=m)
                pc = plsc.all_reduce_population_count(m)
                cs = plsc.cumsum(m.astype(jnp.int32))
                parts.append((s16, dl, m, pc, cs))
            o = off_v
            for s16, dl, m, pc, cs in parts:
                pos = o + cs - 1
                plsc.store_scatter(osrc, [pos], s16, mask=m)
                plsc.store_scatter(odst, [pos], dl, mask=m)
                o = o + pc
            return o

        off_v, tot = ot
        off_v = lax.fori_loop(0, SCCH // 64, group4, off_v)
        return lax.cond(jnp.max(off_v) >= FLUSH, flush,
                        lambda x: x, (off_v, tot))

    stage(0, sva, dva, sema)

    def pair(ip, ot):
        i0 = ip * 2
        stage(i0 + 1, svb, dvb, semb)
        drain(sva, dva, sema)
        ot = compute(sva, dva, ot)

        @pl.when(ip < NSC // 2 - 1)
        def _():
            stage(i0 + 2, sva, dva, sema)

        drain(svb, dvb, semb)
        return compute(svb, dvb, ot)

    off_v, tot = lax.fori_loop(
        0, NSC // 2, pair, (jnp.zeros((16,), jnp.int32), jnp.int32(0)))

    for j in range(IB // 16):
        pos = off_v + j * 16 + iota16
        plsc.store_scatter(osrc, [pos], iota16 + j * 16)
        plsc.store_scatter(odst, [pos],
                           jnp.zeros((16,), jnp.int32) + ROWS_PER_W)
    pltpu.sync_copy(osrc, bsrc_hbm.at[pl.ds(pl.multiple_of(w * EB + tot, 8), OB)])
    pltpu.sync_copy(odst, bdst_hbm.at[pl.ds(pl.multiple_of(w * EB + tot, 8), OB)])
    cbuf[...] = off_v + tot
    pltpu.sync_copy(cbuf, cnt_hbm.at[w])
    pltpu.sync_copy(hist, deg_hbm.at[pl.ds(pl.multiple_of(lo, 8), ROWS_PER_W)])


@functools.partial(
    pl.kernel,
    mesh=_MESH,
    compiler_params=_CP,
    out_type=jax.ShapeDtypeStruct((NP, D), jnp.float32),
    scratch_types=[
        pltpu.VMEM((ROWS_PER_W + 8, D), jnp.float32),
        pltpu.VMEM((IB,), jnp.int32),
        pltpu.VMEM((IB,), jnp.int32),
        pltpu.VMEM((CH, D), jnp.float32),
        pltpu.VMEM((CH, D), jnp.float32),
        pltpu.VMEM((16,), jnp.int32),
        pltpu.SemaphoreType.DMA,
        pltpu.SemaphoreType.DMA,
        pltpu.SemaphoreType.DMA,
    ],
)
def _agg_kernel(g_hbm, bsrc_hbm, bdst_hbm, cnt_hbm, out_hbm,
                acc, sblk, dblk, rowsa, rowsb, cbuf, semi, sema, semb):
    c = lax.axis_index("c")
    s = lax.axis_index("s")
    w = c * NS + s
    base = w * EB

    def zacc(i, carry):
        acc[i >> 3, pl.ds((i & 7) * 16, 16)] = jnp.zeros((16,), jnp.float32)
        return carry

    lax.fori_loop(0, ROWS_PER_W * (D // 16), zacc, 0)

    pltpu.sync_copy(cnt_hbm.at[w], cbuf)
    cnt = jnp.max(cbuf[...])
    nblk = (cnt + IB - 1) >> 10
    iota16 = lax.iota(jnp.int32, 16)

    def gstart(ch, rows, sem):
        pltpu.async_copy(
            g_hbm.at[sblk.at[pl.ds(pl.multiple_of(ch * CH, 8), CH)]],
            rows, sem)

    def gwait(rows, sem):
        pltpu.make_async_copy(g_hbm.at[pl.ds(0, CH)], rows, sem).wait()

    def compute(ch, rows):
        def qq(j, carry2):
            o = ch * CH + j * 32
            dla = dblk[pl.ds(o, 16)]
            dlb = dblk[pl.ds(o + 16, 16)]
            ra = j * 32 + iota16
            rb = ra + 16
            for p in range(D):
                cp = jnp.full((16,), p & ~15, jnp.int32) | (
                    (iota16 + (p & 15)) & 15)
                va = plsc.load_gather(rows, [ra, cp])
                vb = plsc.load_gather(rows, [rb, cp])
                plsc.addupdate_scatter(acc, [dla, cp], va)
                plsc.addupdate_scatter(acc, [dlb, cp], vb)
            return carry2

        lax.fori_loop(0, CH // 32, qq, 0)

    def blk(b, carry):
        o = pl.multiple_of(base + b * IB, 8)
        pltpu.async_copy(bsrc_hbm.at[pl.ds(o, IB)], sblk, semi)
        pltpu.async_copy(bdst_hbm.at[pl.ds(o, IB)], dblk, semi)
        pltpu.make_async_copy(bsrc_hbm.at[pl.ds(0, IB)], sblk, semi).wait()
        pltpu.make_async_copy(bdst_hbm.at[pl.ds(0, IB)], dblk, semi).wait()
        gstart(0, rowsa, sema)

        def pairb(jp, carry2):
            c0 = jp * 2
            gstart(c0 + 1, rowsb, semb)
            gwait(rowsa, sema)
            compute(c0, rowsa)

            @pl.when(jp < (IB // CH) // 2 - 1)
            def _():
                gstart(c0 + 2, rowsa, sema)

            gwait(rowsb, semb)
            compute(c0 + 1, rowsb)
            return carry2

        lax.fori_loop(0, (IB // CH) // 2, pairb, 0)
        return carry

    lax.fori_loop(0, nblk, blk, 0)
    pltpu.sync_copy(acc.at[pl.ds(0, ROWS_PER_W)],
                    out_hbm.at[pl.ds(pl.multiple_of(w * ROWS_PER_W, 8), ROWS_PER_W)])


def _tc0_body(x_ref, w_ref, deg_ref, g_ref, dinv_ref):
    i = pl.program_id(0)
    d = deg_ref[0, pl.ds(i * BR, BR)] + 1.0
    dinv = lax.rsqrt(jnp.maximum(d, 1e-12))
    m = jnp.dot(x_ref[...], w_ref[...], preferred_element_type=jnp.float32)
    g_ref[...] = m * dinv[:, None]
    dinv_ref[0, pl.ds(i * BR, BR)] = dinv


def _tc0(xp, W1, deg):
    return pl.pallas_call(
        _tc0_body,
        grid=(GRID,),
        in_specs=[
            pl.BlockSpec((BR, D), lambda i: (i, 0)),
            pl.BlockSpec((D, D), lambda i: (0, 0)),
            pl.BlockSpec((1, NP), lambda i: (0, 0)),
        ],
        out_specs=[
            pl.BlockSpec((BR, D), lambda i: (i, 0)),
            pl.BlockSpec((1, NP), lambda i: (0, 0)),
        ],
        out_shape=[
            jax.ShapeDtypeStruct((NP, D), jnp.float32),
            jax.ShapeDtypeStruct((1, NP), jnp.float32),
        ],
    )(xp, W1, deg)


def _tcl_body(s_ref, g_ref, dinv_ref, b_ref, w_ref, o_ref):
    i = pl.program_id(0)
    dinv = dinv_ref[0, pl.ds(i * BR, BR)]
    t = (s_ref[...] + g_ref[...]) * dinv[:, None] + b_ref[0, :][None, :]
    h = jnp.maximum(t, 0.0)
    o_ref[...] = jnp.dot(h, w_ref[...],
                         preferred_element_type=jnp.float32) * dinv[:, None]


def _tcl(sp, g, dinv, b, W):
    return pl.pallas_call(
        _tcl_body,
        grid=(GRID,),
        in_specs=[
            pl.BlockSpec((BR, D), lambda i: (i, 0)),
            pl.BlockSpec((BR, D), lambda i: (i, 0)),
            pl.BlockSpec((1, NP), lambda i: (0, 0)),
            pl.BlockSpec((1, D), lambda i: (0, 0)),
            pl.BlockSpec((D, D), lambda i: (0, 0)),
        ],
        out_specs=pl.BlockSpec((BR, D), lambda i: (i, 0)),
        out_shape=jax.ShapeDtypeStruct((NP, D), jnp.float32),
    )(sp, g, dinv, b, W)


def _tcf_body(s_ref, g_ref, dinv_ref, b_ref, o_ref):
    i = pl.program_id(0)
    dinv = dinv_ref[0, pl.ds(i * BR, BR)]
    o_ref[...] = (s_ref[...] + g_ref[...]) * dinv[:, None] \
        + b_ref[0, :][None, :]


def _tcf(sp, g, dinv, b):
    return pl.pallas_call(
        _tcf_body,
        grid=(GRID,),
        in_specs=[
            pl.BlockSpec((BR, D), lambda i: (i, 0)),
            pl.BlockSpec((BR, D), lambda i: (i, 0)),
            pl.BlockSpec((1, NP), lambda i: (0, 0)),
            pl.BlockSpec((1, D), lambda i: (0, 0)),
        ],
        out_specs=pl.BlockSpec((BR, D), lambda i: (i, 0)),
        out_shape=jax.ShapeDtypeStruct((NP, D), jnp.float32),
    )(sp, g, dinv, b)


def kernel(x, edge_index, batch, W1, b1, W2, b2, W3, b3):
    src = edge_index[0]
    dst = edge_index[1]
    xp = jnp.zeros((NP, D), jnp.float32).at[:N].set(x)
    bsrc, bdst, cnt, deg = _bucket_kernel(src, dst)
    g1, dinv = _tc0(xp, W1, deg.reshape(1, NP))
    s1 = _agg_kernel(g1, bsrc, bdst, cnt)
    g2 = _tcl(s1, g1, dinv, b1.reshape(1, D), W2)
    s2 = _agg_kernel(g2, bsrc, bdst, cnt)
    g3 = _tcl(s2, g2, dinv, b2.reshape(1, D), W3)
    s3 = _agg_kernel(g3, bsrc, bdst, cnt)
    out = _tcf(s3, g3, dinv, b3.reshape(1, D))
    return out[:N]
